# baseline (speedup 1.0000x reference)
import functools

import jax
import jax.numpy as jnp
from jax import lax
from jax.experimental import pallas as pl
from jax.experimental.pallas import tpu as pltpu
from jax.experimental.pallas import tpu_sc as plsc

NY, NX, NZ = 128, 128, 128
C = 32
NVOX = 200000
TOTAL = NY * NX * NZ

_info = plsc.get_sparse_core_info()
NC = _info.num_cores
NS = _info.num_subcores
NWORK = NC * NS

VPW = 6256
NVOX_PAD = VPW * NWORK

RANGE = TOTAL // NWORK
CW = 1024
NCHUNK = RANGE // CW
NLCH = 32
LK = NVOX_PAD // NLCH
CAP = 12272
CAPC = 496
NROW = 4

_mesh = plsc.VectorSubcoreMesh(core_axis_name="c", subcore_axis_name="s")
_params = pltpu.CompilerParams(needs_layout_passes=False,
                               use_tc_tiling_on_sc=False)


@functools.partial(
    pl.kernel,
    out_type=jax.ShapeDtypeStruct((NVOX_PAD,), jnp.int32),
    mesh=_mesh,
    compiler_params=_params,
    scratch_types=[
        pltpu.VMEM((VPW * 4,), jnp.int32),
        pltpu.VMEM((VPW,), jnp.int32),
    ],
)
def _lin_kernel(coors_hbm, lin_hbm, cbuf, lbuf):
    wid = lax.axis_index("s") * NC + lax.axis_index("c")
    base = wid * VPW
    pltpu.sync_copy(coors_hbm.at[pl.ds(base * 4, VPW * 4)], cbuf)
    iota = lax.iota(jnp.int32, 16)

    def body(i, _):
        r = i * 16
        rows4 = (iota + r) * 4
        yv = plsc.load_gather(cbuf, [rows4 + 1])
        xv = plsc.load_gather(cbuf, [rows4 + 2])
        zv = plsc.load_gather(cbuf, [rows4 + 3])
        linv = yv * NX + xv + zv * (NX * NY)
        gid = iota + r + base
        linv = jnp.where(gid < NVOX, linv, TOTAL)
        lbuf[pl.ds(r, 16)] = linv
        return 0

    lax.fori_loop(0, VPW // 16, body, 0)
    pltpu.sync_copy(lbuf, lin_hbm.at[pl.ds(base, VPW)])


@functools.partial(
    pl.kernel,
    out_type=jax.ShapeDtypeStruct((C, TOTAL), jnp.float32),
    mesh=_mesh,
    compiler_params=_params,
    scratch_types=[
        pltpu.VMEM((LK,), jnp.int32),
        pltpu.VMEM((LK,), jnp.int32),
        pltpu.VMEM((CAP + 16,), jnp.int32),
        pltpu.VMEM((CAP + 16,), jnp.int32),
        pltpu.VMEM((4 * 512,), jnp.int32),
        pltpu.VMEM((4 * 512,), jnp.int32),
        pltpu.VMEM((NROW, 128), jnp.int32),
        pltpu.VMEM((512, 32), jnp.float32),
        pltpu.VMEM((2, C, CW), jnp.float32),
        pltpu.SemaphoreType.DMA,
        pltpu.SemaphoreType.DMA,
        pltpu.SemaphoreType.DMA,
        pltpu.SemaphoreType.DMA,
    ],
)
def _scatter_kernel(lin_hbm, vf_hbm, out_hbm, lb0, lb1, locs, idsl, clocs,
                    cids, idxb, stage, canvas, sin0, sin1, sg, so):
    wid = lax.axis_index("s") * NC + lax.axis_index("c")
    lo = wid * RANGE
    iota = lax.iota(jnp.int32, 16)
    z16f = jnp.zeros((16,), jnp.float32)
    z16i = jnp.zeros((16,), jnp.int32)

    def ibody(i, _):
        cids[pl.ds(i * 16, 16)] = z16i
        clocs[pl.ds(i * 16, 16)] = z16i
        return 0

    lax.fori_loop(0, (4 * 512) // 16, ibody, 0)

    pltpu.async_copy(lin_hbm.at[pl.ds(0, LK)], lb0, sin0)
    cnt = 0
    for j in range(NLCH):
        buf = lb0 if j % 2 == 0 else lb1
        sem = sin0 if j % 2 == 0 else sin1
        pltpu.make_async_copy(lin_hbm.at[pl.ds(j * LK, LK)], buf, sem).wait()
        if j + 1 < NLCH:
            nbuf = lb1 if j % 2 == 0 else lb0
            nsem = sin1 if j % 2 == 0 else sin0
            pltpu.async_copy(lin_hbm.at[pl.ds((j + 1) * LK, LK)], nbuf, nsem)

        def scan(i, cnt, buf=buf, j=j):
            v = buf[pl.ds(i * 16, 16)]
            m = (v >= lo) & (v < lo + RANGE)
            plsc.store_compressed(locs.at[pl.ds(cnt, 16)], v - lo, mask=m)
            ids = iota + (j * LK + i * 16)
            plsc.store_compressed(idsl.at[pl.ds(cnt, 16)], ids, mask=m)
            return jnp.minimum(cnt + jnp.sum(m.astype(jnp.int32)), CAP)

        cnt = lax.fori_loop(0, LK // 16, scan, cnt)

    nvec = (cnt + 15) // 16

    def zbody(i, _):
        b = i // (C * CW // 16)
        rem = i % (C * CW // 16)
        canvas[b, rem // (CW // 16), pl.ds((rem % (CW // 16)) * 16, 16)] = z16f
        return 0

    lax.fori_loop(0, 2 * C * CW // 16, zbody, 0)

    def chunk_body(ch, carry):
        k1, k2 = carry
        par = ch % 2
        cb = (ch % 4) * 512
        c0 = ch * CW

        def sel(i, k):
            valid = (i * 16 + iota) < cnt
            lv = locs[pl.ds(i * 16, 16)]
            dv = lv - c0
            m = valid & (dv >= 0) & (dv < CW)
            plsc.store_compressed(clocs.at[pl.ds(cb + k, 16)], dv, mask=m)
            idv = idsl[pl.ds(i * 16, 16)]
            plsc.store_compressed(cids.at[pl.ds(cb + k, 16)], idv, mask=m)
            return jnp.minimum(k + jnp.sum(m.astype(jnp.int32)), CAPC)

        k = lax.fori_loop(0, nvec, sel, 0)

        def cpy(i, _):
            idxb[i // 8, pl.ds((i % 8) * 16, 16)] = cids[pl.ds(cb + i * 16, 16)]
            return 0

        lax.fori_loop(0, NROW * 8, cpy, 0)

        nrow = (k + 127) // 128

        def gat(r, _):
            pltpu.async_copy(vf_hbm.at[idxb.at[r]],
                             stage.at[pl.ds(r * 128, 128), :], sg)
            return 0

        lax.fori_loop(0, nrow, gat, 0)

        def gwait(r, _):
            pltpu.make_async_copy(vf_hbm.at[idxb.at[0]],
                                  stage.at[pl.ds(0, 128), :], sg).wait()
            return 0

        lax.fori_loop(0, nrow, gwait, 0)

        @pl.when(ch >= 2)
        def _():
            pltpu.make_async_copy(canvas.at[par], out_hbm.at[:, pl.ds(lo, CW)],
                                  so).wait()
            pb = ((ch - 2) % 4) * 512

            def rz(q, _):
                b16 = q * 16
                locv = clocs[pl.ds(pb + b16, 16)]
                for l in range(16):
                    m = jnp.full((16,), (b16 + l) < k2)
                    locl = jnp.full((16,), locv[l], jnp.int32)
                    plsc.store_scatter(canvas.at[par], [iota, locl], z16f,
                                       mask=m)
                    plsc.store_scatter(canvas.at[par], [iota + 16, locl],
                                       z16f, mask=m)
                return 0

            lax.fori_loop(0, (k2 + 15) // 16, rz, 0)

        def sc(q, _):
            b16 = q * 16
            locv = clocs[pl.ds(cb + b16, 16)]
            for l in range(16):
                m = jnp.full((16,), (b16 + l) < k)
                locl = jnp.full((16,), locv[l], jnp.int32)
                v0 = stage[b16 + l, pl.ds(0, 16)]
                v1 = stage[b16 + l, pl.ds(16, 16)]
                plsc.store_scatter(canvas.at[par], [iota, locl], v0, mask=m)
                plsc.store_scatter(canvas.at[par], [iota + 16, locl], v1,
                                   mask=m)
            return 0

        lax.fori_loop(0, (k + 15) // 16, sc, 0)

        pltpu.async_copy(canvas.at[par], out_hbm.at[:, pl.ds(lo + c0, CW)], so)
        return (k, k1)

    lax.fori_loop(0, NCHUNK, chunk_body, (0, 0))
    pltpu.make_async_copy(canvas.at[0], out_hbm.at[:, pl.ds(lo, CW)],
                          so).wait()
    pltpu.make_async_copy(canvas.at[1], out_hbm.at[:, pl.ds(lo, CW)],
                          so).wait()


def kernel(voxel_features, coors):
    coors_p = jnp.pad(coors, ((0, NVOX_PAD - NVOX), (0, 0))).reshape(-1)
    lin = _lin_kernel(coors_p)
    canvas = _scatter_kernel(lin, voxel_features)
    return canvas.reshape(1, C, NY, NX, NZ)


# speedup vs baseline: 1.0518x; 1.0518x over previous
import functools

import jax
import jax.numpy as jnp
from jax import lax
from jax.experimental import pallas as pl
from jax.experimental.pallas import tpu as pltpu
from jax.experimental.pallas import tpu_sc as plsc

NY, NX, NZ = 128, 128, 128
C = 32
NVOX = 200000
TOTAL = NY * NX * NZ

_info = plsc.get_sparse_core_info()
NC = _info.num_cores
NS = _info.num_subcores
NWORK = NC * NS

VPW = 6256
NVOX_PAD = VPW * NWORK

RANGE = TOTAL // NWORK
CW = 1024
NCHUNK = RANGE // CW
NLCH = 16
LK = NVOX_PAD // NLCH
CAP = 16368
CAPC = 496
NROW = 4

_mesh = plsc.VectorSubcoreMesh(core_axis_name="c", subcore_axis_name="s")
_params = pltpu.CompilerParams(needs_layout_passes=False,
                               use_tc_tiling_on_sc=False)


@functools.partial(
    pl.kernel,
    out_type=jax.ShapeDtypeStruct((NVOX_PAD,), jnp.int32),
    mesh=_mesh,
    compiler_params=_params,
    scratch_types=[
        pltpu.VMEM((VPW * 4,), jnp.int32),
        pltpu.VMEM((VPW,), jnp.int32),
    ],
)
def _lin_kernel(coors_hbm, lin_hbm, cbuf, lbuf):
    wid = lax.axis_index("s") * NC + lax.axis_index("c")
    base = wid * VPW
    pltpu.sync_copy(coors_hbm.at[pl.ds(base * 4, VPW * 4)], cbuf)
    iota = lax.iota(jnp.int32, 16)

    def body(i, _):
        r = i * 16
        rows4 = (iota + r) * 4
        yv = plsc.load_gather(cbuf, [rows4 + 1])
        xv = plsc.load_gather(cbuf, [rows4 + 2])
        zv = plsc.load_gather(cbuf, [rows4 + 3])
        linv = yv * NX + xv + zv * (NX * NY)
        gid = iota + r + base
        linv = jnp.where(gid < NVOX, linv, TOTAL)
        lbuf[pl.ds(r, 16)] = linv
        return 0

    lax.fori_loop(0, VPW // 16, body, 0)
    pltpu.sync_copy(lbuf, lin_hbm.at[pl.ds(base, VPW)])


@functools.partial(
    pl.kernel,
    out_type=jax.ShapeDtypeStruct((C, TOTAL), jnp.float32),
    mesh=_mesh,
    compiler_params=_params,
    scratch_types=[
        pltpu.VMEM((LK,), jnp.int32),
        pltpu.VMEM((LK,), jnp.int32),
        pltpu.VMEM((CAP + 16,), jnp.int32),
        pltpu.VMEM((CAP + 16,), jnp.int32),
        pltpu.VMEM((2 * 512,), jnp.int32),
        pltpu.VMEM((2 * 512,), jnp.int32),
        pltpu.VMEM((NROW, 128), jnp.int32),
        pltpu.VMEM((512, 32), jnp.float32),
        pltpu.VMEM((C, CW), jnp.float32),
        pltpu.SemaphoreType.DMA,
        pltpu.SemaphoreType.DMA,
        pltpu.SemaphoreType.DMA,
        pltpu.SemaphoreType.DMA,
    ],
)
def _scatter_kernel(lin_hbm, vf_hbm, out_hbm, lb0, lb1, locs, idsl, clocs,
                    cids, idxb, stage, canvas, sin0, sin1, sg, so):
    wid = lax.axis_index("s") * NC + lax.axis_index("c")
    lo = wid * RANGE
    iota = lax.iota(jnp.int32, 16)
    z16f = jnp.zeros((16,), jnp.float32)
    z16i = jnp.zeros((16,), jnp.int32)

    def ibody(i, _):
        cids[pl.ds(i * 16, 16)] = z16i
        clocs[pl.ds(i * 16, 16)] = z16i
        return 0

    lax.fori_loop(0, (2 * 512) // 16, ibody, 0)

    pltpu.async_copy(lin_hbm.at[pl.ds(0, LK)], lb0, sin0)
    cnt = 0
    for j in range(NLCH):
        buf = lb0 if j % 2 == 0 else lb1
        sem = sin0 if j % 2 == 0 else sin1
        pltpu.make_async_copy(lin_hbm.at[pl.ds(j * LK, LK)], buf, sem).wait()
        if j + 1 < NLCH:
            nbuf = lb1 if j % 2 == 0 else lb0
            nsem = sin1 if j % 2 == 0 else sin0
            pltpu.async_copy(lin_hbm.at[pl.ds((j + 1) * LK, LK)], nbuf, nsem)

        def scan(i, cnt, buf=buf, j=j):
            v = buf[pl.ds(i * 16, 16)]
            m = (v >= lo) & (v < lo + RANGE)
            plsc.store_compressed(locs.at[pl.ds(cnt, 16)], v - lo, mask=m)
            ids = iota + (j * LK + i * 16)
            plsc.store_compressed(idsl.at[pl.ds(cnt, 16)], ids, mask=m)
            return jnp.minimum(cnt + jnp.sum(m.astype(jnp.int32)), CAP)

        cnt = lax.fori_loop(0, LK // 16, scan, cnt)

    nvec = (cnt + 15) // 16

    def zbody(i, _):
        canvas[i // (CW // 16), pl.ds((i % (CW // 16)) * 16, 16)] = z16f
        return 0

    lax.fori_loop(0, C * CW // 16, zbody, 0)

    def chunk_body(ch, kprev):
        par = ch % 2
        cb = par * 512
        c0 = ch * CW

        def sel(i, k):
            valid = (i * 16 + iota) < cnt
            lv = locs[pl.ds(i * 16, 16)]
            dv = lv - c0
            m = valid & (dv >= 0) & (dv < CW)
            plsc.store_compressed(clocs.at[pl.ds(cb + k, 16)], dv, mask=m)
            idv = idsl[pl.ds(i * 16, 16)]
            plsc.store_compressed(cids.at[pl.ds(cb + k, 16)], idv, mask=m)
            return jnp.minimum(k + jnp.sum(m.astype(jnp.int32)), CAPC)

        k = lax.fori_loop(0, nvec, sel, 0)

        def cpy(i, _):
            idxb[i // 8, pl.ds((i % 8) * 16, 16)] = cids[pl.ds(cb + i * 16, 16)]
            return 0

        lax.fori_loop(0, NROW * 8, cpy, 0)

        nrow = (k + 127) // 128

        def gat(r, _):
            pltpu.async_copy(vf_hbm.at[idxb.at[r]],
                             stage.at[pl.ds(r * 128, 128), :], sg)
            return 0

        lax.fori_loop(0, nrow, gat, 0)

        def gwait(r, _):
            pltpu.make_async_copy(vf_hbm.at[idxb.at[0]],
                                  stage.at[pl.ds(0, 128), :], sg).wait()
            return 0

        lax.fori_loop(0, nrow, gwait, 0)

        @pl.when(ch > 0)
        def _():
            pltpu.make_async_copy(canvas, out_hbm.at[:, pl.ds(lo, CW)],
                                  so).wait()
            pb = (1 - par) * 512

            def rz(q, _):
                b16 = q * 16
                locv = clocs[pl.ds(pb + b16, 16)]
                for l in range(16):
                    m = jnp.full((16,), (b16 + l) < kprev)
                    locl = jnp.full((16,), locv[l], jnp.int32)
                    plsc.store_scatter(canvas, [iota, locl], z16f, mask=m)
                    plsc.store_scatter(canvas, [iota + 16, locl], z16f, mask=m)
                return 0

            lax.fori_loop(0, (kprev + 15) // 16, rz, 0)

        def sc(q, _):
            b16 = q * 16
            locv = clocs[pl.ds(cb + b16, 16)]
            for l in range(16):
                m = jnp.full((16,), (b16 + l) < k)
                locl = jnp.full((16,), locv[l], jnp.int32)
                v0 = stage[b16 + l, pl.ds(0, 16)]
                v1 = stage[b16 + l, pl.ds(16, 16)]
                plsc.store_scatter(canvas, [iota, locl], v0, mask=m)
                plsc.store_scatter(canvas, [iota + 16, locl], v1, mask=m)
            return 0

        lax.fori_loop(0, (k + 15) // 16, sc, 0)

        pltpu.async_copy(canvas, out_hbm.at[:, pl.ds(lo + c0, CW)], so)
        return k

    lax.fori_loop(0, NCHUNK, chunk_body, 0)
    pltpu.make_async_copy(canvas, out_hbm.at[:, pl.ds(lo, CW)], so).wait()


def kernel(voxel_features, coors):
    coors_p = jnp.pad(coors, ((0, NVOX_PAD - NVOX), (0, 0))).reshape(-1)
    lin = _lin_kernel(coors_p)
    canvas = _scatter_kernel(lin, voxel_features)
    return canvas.reshape(1, C, NY, NX, NZ)


# speedup vs baseline: 1.0676x; 1.0150x over previous
import functools

import jax
import jax.numpy as jnp
from jax import lax
from jax.experimental import pallas as pl
from jax.experimental.pallas import tpu as pltpu
from jax.experimental.pallas import tpu_sc as plsc

NY, NX, NZ = 128, 128, 128
C = 32
NVOX = 200000
TOTAL = NY * NX * NZ

_info = plsc.get_sparse_core_info()
NC = _info.num_cores
NS = _info.num_subcores
NWORK = NC * NS

VPW = 6256
NVOX_PAD = VPW * NWORK

RANGE = TOTAL // NWORK
CW = 1024
NCHUNK = RANGE // CW
NLCH = 16
LK = NVOX_PAD // NLCH
CAP = 16368
CAPC = 496
NROW = 4

_mesh = plsc.VectorSubcoreMesh(core_axis_name="c", subcore_axis_name="s")
_params = pltpu.CompilerParams(needs_layout_passes=False,
                               use_tc_tiling_on_sc=False)


@functools.partial(
    pl.kernel,
    out_type=jax.ShapeDtypeStruct((NVOX_PAD,), jnp.int32),
    mesh=_mesh,
    compiler_params=_params,
    scratch_types=[
        pltpu.VMEM((VPW * 4,), jnp.int32),
        pltpu.VMEM((VPW,), jnp.int32),
    ],
)
def _lin_kernel(coors_hbm, lin_hbm, cbuf, lbuf):
    wid = lax.axis_index("s") * NC + lax.axis_index("c")
    base = wid * VPW
    pltpu.sync_copy(coors_hbm.at[pl.ds(base * 4, VPW * 4)], cbuf)
    iota = lax.iota(jnp.int32, 16)

    def body(i, _):
        r = i * 16
        rows4 = (iota + r) * 4
        yv = plsc.load_gather(cbuf, [rows4 + 1])
        xv = plsc.load_gather(cbuf, [rows4 + 2])
        zv = plsc.load_gather(cbuf, [rows4 + 3])
        linv = yv * NX + xv + zv * (NX * NY)
        gid = iota + r + base
        linv = jnp.where(gid < NVOX, linv, TOTAL)
        lbuf[pl.ds(r, 16)] = linv
        return 0

    lax.fori_loop(0, VPW // 16, body, 0)
    pltpu.sync_copy(lbuf, lin_hbm.at[pl.ds(base, VPW)])


@functools.partial(
    pl.kernel,
    out_type=jax.ShapeDtypeStruct((C, TOTAL), jnp.float32),
    mesh=_mesh,
    compiler_params=_params,
    scratch_types=[
        pltpu.VMEM((LK,), jnp.int32),
        pltpu.VMEM((LK,), jnp.int32),
        pltpu.VMEM((CAP + 16,), jnp.int32),
        pltpu.VMEM((CAP + 16,), jnp.int32),
        pltpu.VMEM((2 * 512,), jnp.int32),
        pltpu.VMEM((2 * 512,), jnp.int32),
        pltpu.VMEM((NROW, 128), jnp.int32),
        pltpu.VMEM((512, 32), jnp.float32),
        pltpu.VMEM((C, CW), jnp.float32),
        pltpu.SemaphoreType.DMA,
        pltpu.SemaphoreType.DMA,
        pltpu.SemaphoreType.DMA,
        pltpu.SemaphoreType.DMA,
    ],
)
def _scatter_kernel(lin_hbm, vf_hbm, out_hbm, lb0, lb1, locs, idsl, clocs,
                    cids, idxb, stage, canvas, sin0, sin1, sg, so):
    wid = lax.axis_index("s") * NC + lax.axis_index("c")
    lo = wid * RANGE
    iota = lax.iota(jnp.int32, 16)
    z16f = jnp.zeros((16,), jnp.float32)
    z16i = jnp.zeros((16,), jnp.int32)

    def ibody(i, _):
        cids[pl.ds(i * 16, 16)] = z16i
        clocs[pl.ds(i * 16, 16)] = z16i
        return 0

    lax.fori_loop(0, (2 * 512) // 16, ibody, 0)

    pltpu.async_copy(lin_hbm.at[pl.ds(0, LK)], lb0, sin0)
    cnt = 0
    for j in range(NLCH):
        buf = lb0 if j % 2 == 0 else lb1
        sem = sin0 if j % 2 == 0 else sin1
        pltpu.make_async_copy(lin_hbm.at[pl.ds(j * LK, LK)], buf, sem).wait()
        if j + 1 < NLCH:
            nbuf = lb1 if j % 2 == 0 else lb0
            nsem = sin1 if j % 2 == 0 else sin0
            pltpu.async_copy(lin_hbm.at[pl.ds((j + 1) * LK, LK)], nbuf, nsem)

        def scan(i, cnt, buf=buf, j=j):
            v = buf[pl.ds(i * 16, 16)]
            m = (v >= lo) & (v < lo + RANGE)
            plsc.store_compressed(locs.at[pl.ds(cnt, 16)], v - lo, mask=m)
            ids = iota + (j * LK + i * 16)
            plsc.store_compressed(idsl.at[pl.ds(cnt, 16)], ids, mask=m)
            pc = plsc.all_reduce_population_count(m)
            return jnp.minimum(cnt + pc[0], CAP)

        cnt = lax.fori_loop(0, LK // 16, scan, cnt)

    nvec = (cnt + 15) // 16

    def zbody(i, _):
        canvas[i // (CW // 16), pl.ds((i % (CW // 16)) * 16, 16)] = z16f
        return 0

    lax.fori_loop(0, C * CW // 16, zbody, 0)

    def chunk_body(ch, kprev):
        par = ch % 2
        cb = par * 512
        c0 = ch * CW

        def sel(i, k):
            valid = (i * 16 + iota) < cnt
            lv = locs[pl.ds(i * 16, 16)]
            dv = lv - c0
            m = valid & (dv >= 0) & (dv < CW)
            plsc.store_compressed(clocs.at[pl.ds(cb + k, 16)], dv, mask=m)
            idv = idsl[pl.ds(i * 16, 16)]
            plsc.store_compressed(cids.at[pl.ds(cb + k, 16)], idv, mask=m)
            pc = plsc.all_reduce_population_count(m)
            return jnp.minimum(k + pc[0], CAPC)

        k = lax.fori_loop(0, nvec, sel, 0)

        def cpy(i, _):
            idxb[i // 8, pl.ds((i % 8) * 16, 16)] = cids[pl.ds(cb + i * 16, 16)]
            return 0

        lax.fori_loop(0, NROW * 8, cpy, 0)

        nrow = (k + 127) // 128

        def gat(r, _):
            pltpu.async_copy(vf_hbm.at[idxb.at[r]],
                             stage.at[pl.ds(r * 128, 128), :], sg)
            return 0

        lax.fori_loop(0, nrow, gat, 0)

        def gwait(r, _):
            pltpu.make_async_copy(vf_hbm.at[idxb.at[0]],
                                  stage.at[pl.ds(0, 128), :], sg).wait()
            return 0

        lax.fori_loop(0, nrow, gwait, 0)

        @pl.when(ch > 0)
        def _():
            pltpu.make_async_copy(canvas, out_hbm.at[:, pl.ds(lo, CW)],
                                  so).wait()
            pb = (1 - par) * 512

            def rz(q, _):
                b16 = q * 16
                locv = clocs[pl.ds(pb + b16, 16)]
                for l in range(16):
                    m = jnp.full((16,), (b16 + l) < kprev)
                    locl = jnp.full((16,), locv[l], jnp.int32)
                    plsc.store_scatter(canvas, [iota, locl], z16f, mask=m)
                    plsc.store_scatter(canvas, [iota + 16, locl], z16f, mask=m)
                return 0

            lax.fori_loop(0, (kprev + 15) // 16, rz, 0)

        def sc(q, _):
            b16 = q * 16
            locv = clocs[pl.ds(cb + b16, 16)]
            for l in range(16):
                m = jnp.full((16,), (b16 + l) < k)
                locl = jnp.full((16,), locv[l], jnp.int32)
                v0 = stage[b16 + l, pl.ds(0, 16)]
                v1 = stage[b16 + l, pl.ds(16, 16)]
                plsc.store_scatter(canvas, [iota, locl], v0, mask=m)
                plsc.store_scatter(canvas, [iota + 16, locl], v1, mask=m)
            return 0

        lax.fori_loop(0, (k + 15) // 16, sc, 0)

        pltpu.async_copy(canvas, out_hbm.at[:, pl.ds(lo + c0, CW)], so)
        return k

    lax.fori_loop(0, NCHUNK, chunk_body, 0)
    pltpu.make_async_copy(canvas, out_hbm.at[:, pl.ds(lo, CW)], so).wait()


def kernel(voxel_features, coors):
    coors_p = jnp.pad(coors, ((0, NVOX_PAD - NVOX), (0, 0))).reshape(-1)
    lin = _lin_kernel(coors_p)
    canvas = _scatter_kernel(lin, voxel_features)
    return canvas.reshape(1, C, NY, NX, NZ)
